# baseline (speedup 1.0000x reference)
import jax
import jax.numpy as jnp
from jax import lax
from jax.experimental import pallas as pl
from jax.experimental.pallas import tpu as pltpu
from jax.experimental.pallas import tpu_sc as plsc

N = 1024
U = 200
D = 64
VECS = D // 16

_info = plsc.get_sparse_core_info()
NC, NS = _info.num_cores, _info.num_subcores
NW = NC * NS
SEQ_PER_W = N // NW

CH0 = 128
CH1 = U - CH0


def _sc_body(y_hbm, table_hbm, w_hbm, out_hbm, idx_v, rows_v, outb_v, w_v, sem):
    wid = lax.axis_index("s") * NC + lax.axis_index("c")

    pltpu.sync_copy(w_hbm, w_v)
    w0 = [w_v[0, pl.ds(16 * j, 16)] for j in range(VECS)]
    w1 = [w_v[1, pl.ds(16 * j, 16)] for j in range(VECS)]
    zero = jnp.zeros((16,), jnp.float32)

    def seq_body(s_i, carry):
        base = (wid * SEQ_PER_W + s_i) * U
        pltpu.sync_copy(y_hbm.at[pl.ds(base, U)], idx_v)
        cp0 = pltpu.async_copy(
            table_hbm.at[idx_v.at[pl.ds(0, CH0)]], rows_v.at[pl.ds(0, CH0)], sem)
        cp1 = pltpu.async_copy(
            table_hbm.at[idx_v.at[pl.ds(CH0, CH1)]], rows_v.at[pl.ds(CH0, CH1)], sem)
        cp0.wait()
        cp1.wait()

        def u_body(u, prev):
            cur = tuple(rows_v[u, pl.ds(16 * j, 16)] for j in range(VECS))
            for j in range(VECS):
                outb_v[u, pl.ds(16 * j, 16)] = jnp.maximum(
                    prev[j] * w0[j] + cur[j] * w1[j], 0.0)
            return cur

        lax.fori_loop(0, U, u_body, (zero,) * VECS)
        pltpu.sync_copy(outb_v, out_hbm.at[pl.ds(base, U)])
        return carry

    lax.fori_loop(0, SEQ_PER_W, seq_body, 0)


_sc_call = pl.kernel(
    _sc_body,
    out_type=jax.ShapeDtypeStruct((N * U, D), jnp.float32),
    mesh=plsc.VectorSubcoreMesh(core_axis_name="c", subcore_axis_name="s"),
    scratch_types=[
        pltpu.VMEM((U,), jnp.int32),
        pltpu.VMEM((U, D), jnp.float32),
        pltpu.VMEM((U, D), jnp.float32),
        pltpu.VMEM((2, D), jnp.float32),
        pltpu.SemaphoreType.DMA,
    ],
    compiler_params=pltpu.CompilerParams(use_tc_tiling_on_sc=False),
)


@jax.jit
def kernel(y, table, conv_w):
    y_flat = y.reshape(N * U).astype(jnp.int32)
    w = conv_w.T
    out = _sc_call(y_flat, table, w)
    return out.reshape(N, U, D)
